# baseline (speedup 1.0000x reference)
import jax
import jax.numpy as jnp
from jax import lax
from jax.experimental import pallas as pl
from jax.experimental.pallas import tpu as pltpu
from jax.experimental.pallas import tpu_sc as plsc

N = 10000
E = 320000
D = 128
G = 64
H1 = 256
OUT = 10
L = 3

NC = 2
NS = 16
NW = NC * NS
EPW = E // NW
C = 40
NCH = EPW // C
N_PAD = 10240
ROWS_PER_TILE = N_PAD // NS
NSTAGE = 5
NJ = NCH // NSTAGE

EPS = 1e-7


NP = NJ // 2


def _sc_body(h_hbm, e_hbm, src_hbm, dst_hbm, out_hbm,
             src_v, dst_v, hb0, hb1, eb0, eb1, acc,
             g0, g1, es0, es1, ss0, ss1):
    c = lax.axis_index("c")
    s = lax.axis_index("s")
    wid = c * NS + s
    hbufs = (hb0, hb1)
    ebufs = (eb0, eb1)
    gsems = (g0, g1)
    esems = (es0, es1)
    ssems = (ss0, ss1)

    def zrow(r, _):
        zero = jnp.zeros((16,), jnp.float32)
        for cc in range(8):
            hb0[r, pl.ds(cc * 16, 16)] = zero
        return 0
    lax.fori_loop(0, 2 * C, zrow, 0)
    for k in range(ROWS_PER_TILE // (2 * C)):
        pltpu.sync_copy(hb0, acc.at[pl.ds(s * ROWS_PER_TILE + k * 2 * C, 2 * C)])
    plsc.subcore_barrier()

    ebase = wid * EPW

    def compute(hbuf, rowoff, ebuf):
        def row(r, _):
            for cc in range(8):
                sl = pl.ds(cc * 16, 16)
                hbuf[rowoff + r, sl] = jnp.maximum(
                    hbuf[rowoff + r, sl] + ebuf[r, sl], EPS)
            return 0
        lax.fori_loop(0, C, row, 0)

    def stage(st, _):
        pltpu.sync_copy(src_hbm.at[wid, st], src_v)
        pltpu.sync_copy(dst_hbm.at[wid, st], dst_v)
        gp = [None] * NP
        ed = [None] * NJ
        sd = [None] * NJ
        gp[0] = pltpu.async_copy(h_hbm.at[src_v.at[0]], hb0, g0)
        ed[0] = pltpu.async_copy(e_hbm.at[pl.ds((ebase + st * NJ * C), C)],
                                 eb0, es0)
        for p in range(NP):
            pb = p & 1
            if p + 1 < NP:
                if p >= 1:
                    sd[2 * (p - 1)].wait()
                    sd[2 * (p - 1) + 1].wait()
                gp[p + 1] = pltpu.async_copy(h_hbm.at[src_v.at[p + 1]],
                                             hbufs[1 - pb], gsems[1 - pb])
            gp[p].wait()
            for u in range(2):
                j = 2 * p + u
                if j + 1 < NJ:
                    ed[j + 1] = pltpu.async_copy(
                        e_hbm.at[pl.ds(ebase + (st * NJ + j + 1) * C, C)],
                        ebufs[(j + 1) & 1], esems[(j + 1) & 1])
                ed[j].wait()
                compute(hbufs[pb], u * C, ebufs[j & 1])
                sd[j] = pltpu.async_copy(hbufs[pb].at[pl.ds(u * C, C)],
                                         acc.at[dst_v.at[j]],
                                         ssems[j & 1], add=True)
        for q in range(4):
            sd[NJ - 1 - q].wait()
        return 0
    lax.fori_loop(0, NSTAGE, stage, 0)

    plsc.subcore_barrier()
    pltpu.sync_copy(acc.at[pl.ds(s * ROWS_PER_TILE, ROWS_PER_TILE)],
                    out_hbm.at[pl.ds(c * N_PAD + s * ROWS_PER_TILE, ROWS_PER_TILE)])


@jax.jit
def _sc_msg_agg(h, e_eps, src_r, dst_r):
    mesh = plsc.VectorSubcoreMesh(core_axis_name="c", subcore_axis_name="s",
                                  num_cores=NC, num_subcores=NS)
    return pl.kernel(
        _sc_body,
        out_type=jax.ShapeDtypeStruct((NC * N_PAD, D), jnp.float32),
        mesh=mesh,
        scratch_types=[
            pltpu.VMEM((NP, 2 * C), jnp.int32),
            pltpu.VMEM((NJ, C), jnp.int32),
            pltpu.VMEM((2 * C, D), jnp.float32),
            pltpu.VMEM((2 * C, D), jnp.float32),
            pltpu.VMEM((C, D), jnp.float32),
            pltpu.VMEM((C, D), jnp.float32),
            pltpu.VMEM_SHARED((N_PAD, D), jnp.float32),
            pltpu.SemaphoreType.DMA,
            pltpu.SemaphoreType.DMA,
            pltpu.SemaphoreType.DMA,
            pltpu.SemaphoreType.DMA,
            pltpu.SemaphoreType.DMA,
            pltpu.SemaphoreType.DMA,
        ],
    )(h, e_eps, src_r, dst_r)


_EBLK = 2000
_HBLK = 1024


def _enc_both_body(ea_ref, we_ref, be_ref, x_ref, wn_ref, bn_ref,
                   e_ref, h_ref):
    i = pl.program_id(0)
    e_ref[...] = jnp.dot(ea_ref[...], we_ref[...],
                         preferred_element_type=jnp.float32) + be_ref[...]

    @pl.when(i < N_PAD // _HBLK)
    def _node():
        h_ref[...] = jnp.dot(x_ref[...], wn_ref[...],
                             preferred_element_type=jnp.float32) + bn_ref[...]


def _tc_encode_both(edge_attr, we, be, x_pad, wn, bn):
    k_e = edge_attr.shape[1]
    nh = N_PAD // _HBLK
    return pl.pallas_call(
        _enc_both_body,
        grid=(E // _EBLK,),
        in_specs=[
            pl.BlockSpec((_EBLK, k_e), lambda i: (i, 0)),
            pl.BlockSpec((k_e, D), lambda i: (0, 0)),
            pl.BlockSpec((1, D), lambda i: (0, 0)),
            pl.BlockSpec((_HBLK, D), lambda i: (jnp.minimum(i, nh - 1), 0)),
            pl.BlockSpec((D, D), lambda i: (0, 0)),
            pl.BlockSpec((1, D), lambda i: (0, 0)),
        ],
        out_specs=[
            pl.BlockSpec((_EBLK, D), lambda i: (i, 0)),
            pl.BlockSpec((_HBLK, D), lambda i: (jnp.minimum(i, nh - 1), 0)),
        ],
        out_shape=[jax.ShapeDtypeStruct((E, D), jnp.float32),
                   jax.ShapeDtypeStruct((N_PAD, D), jnp.float32)],
    )(edge_attr, we, (be + EPS).reshape(1, D), x_pad, wn, bn.reshape(1, D))


def _mlp_body(p_ref, h_ref, w1_ref, b1_ref, w2_ref, b2_ref, o_ref):
    a = p_ref[0] + p_ref[1] + h_ref[...]
    mid = jnp.maximum(jnp.dot(a, w1_ref[...],
                              preferred_element_type=jnp.float32) + b1_ref[...], 0.0)
    o = jnp.dot(mid, w2_ref[...], preferred_element_type=jnp.float32) + b2_ref[...]
    o_ref[...] = jnp.maximum(o, 0.0)


def _tc_mlp(parts, h, w1, b1, w2, b2):
    blk = 1024
    return pl.pallas_call(
        _mlp_body,
        grid=(N_PAD // blk,),
        in_specs=[
            pl.BlockSpec((NC, blk, D), lambda i: (0, i, 0)),
            pl.BlockSpec((blk, D), lambda i: (i, 0)),
            pl.BlockSpec((D, H1), lambda i: (0, 0)),
            pl.BlockSpec((1, H1), lambda i: (0, 0)),
            pl.BlockSpec((H1, D), lambda i: (0, 0)),
            pl.BlockSpec((1, D), lambda i: (0, 0)),
        ],
        out_specs=pl.BlockSpec((blk, D), lambda i: (i, 0)),
        out_shape=jax.ShapeDtypeStruct((N_PAD, D), jnp.float32),
    )(parts, h, w1, b1.reshape(1, H1), w2, b2.reshape(1, D))


def _mlp_head_body(p_ref, h_ref, w1_ref, b1_ref, w2_ref, b2_ref, b3_ref,
                   wd1_ref, bd1_ref, wo_ref, bo_ref, o_ref, sums, counts):
    i = pl.program_id(0)
    nb = pl.num_programs(0)

    @pl.when(i == 0)
    def _init():
        sums[...] = jnp.zeros_like(sums)
        counts[...] = jnp.zeros_like(counts)

    a = p_ref[0] + p_ref[1] + h_ref[...]
    mid = jnp.maximum(jnp.dot(a, w1_ref[...],
                              preferred_element_type=jnp.float32) + b1_ref[...], 0.0)
    o = jnp.dot(mid, w2_ref[...], preferred_element_type=jnp.float32) + b2_ref[...]
    o = jnp.maximum(o, 0.0)

    bblk = b3_ref[0]
    gids = lax.broadcasted_iota(jnp.int32, (G, bblk.shape[1]), 0)
    oh = (gids == bblk).astype(jnp.float32)
    sums[...] += jnp.dot(oh, o, preferred_element_type=jnp.float32)
    counts[...] += jnp.dot(oh, jnp.ones_like(o),
                           preferred_element_type=jnp.float32)

    @pl.when(i == nb - 1)
    def _fin():
        pooled = sums[...] / jnp.maximum(counts[...], 1.0)
        z = jnp.maximum(jnp.dot(pooled, wd1_ref[...],
                                preferred_element_type=jnp.float32) + bd1_ref[...], 0.0)
        o_ref[...] = jnp.dot(z, wo_ref[...],
                             preferred_element_type=jnp.float32) + bo_ref[...]


def _tc_mlp_head(parts, h, w1, b1, w2, b2, batch3, wd1, bd1, wo_p, bo_p):
    blk = 1024
    return pl.pallas_call(
        _mlp_head_body,
        grid=(N_PAD // blk,),
        in_specs=[
            pl.BlockSpec((NC, blk, D), lambda i: (0, i, 0)),
            pl.BlockSpec((blk, D), lambda i: (i, 0)),
            pl.BlockSpec((D, H1), lambda i: (0, 0)),
            pl.BlockSpec((1, H1), lambda i: (0, 0)),
            pl.BlockSpec((H1, D), lambda i: (0, 0)),
            pl.BlockSpec((1, D), lambda i: (0, 0)),
            pl.BlockSpec((1, 1, blk), lambda i: (i, 0, 0)),
            pl.BlockSpec((D, H1), lambda i: (0, 0)),
            pl.BlockSpec((1, H1), lambda i: (0, 0)),
            pl.BlockSpec((H1, D), lambda i: (0, 0)),
            pl.BlockSpec((1, D), lambda i: (0, 0)),
        ],
        out_specs=pl.BlockSpec((G, D), lambda i: (0, 0)),
        out_shape=jax.ShapeDtypeStruct((G, D), jnp.float32),
        scratch_shapes=[
            pltpu.VMEM((G, D), jnp.float32),
            pltpu.VMEM((G, D), jnp.float32),
        ],
    )(parts, h, w1, b1.reshape(1, H1), w2, b2.reshape(1, D), batch3,
      wd1, bd1.reshape(1, H1), wo_p, bo_p)


def kernel(x, edge_index, edge_attr, batch, W_node, b_node, W_edge, b_edge,
           conv_W1, conv_b1, conv_W2, conv_b2, W_d1, b_d1, W_out, b_out):
    src_r = edge_index[0].reshape(NW, NSTAGE, NJ // 2, 2 * C)
    dst_r = edge_index[1].reshape(NW, NSTAGE, NJ, C)
    x_pad = jnp.zeros((N_PAD, D), jnp.float32).at[:N].set(x)
    batch_pad = jnp.concatenate(
        [batch, jnp.full((N_PAD - N,), G, jnp.int32)])
    batch3 = batch_pad.reshape(N_PAD // 1024, 1, 1024)
    wo_p = jnp.zeros((H1, D), jnp.float32).at[:, :OUT].set(W_out)
    bo_p = jnp.zeros((1, D), jnp.float32).at[:, :OUT].set(b_out)

    e, h = _tc_encode_both(edge_attr, W_edge, b_edge, x_pad, W_node, b_node)
    for i in range(L):
        parts = _sc_msg_agg(h, e, src_r, dst_r).reshape(NC, N_PAD, D)
        if i < L - 1:
            h = _tc_mlp(parts, h, conv_W1[i], conv_b1[i],
                        conv_W2[i], conv_b2[i])
        else:
            out = _tc_mlp_head(parts, h, conv_W1[i], conv_b1[i],
                               conv_W2[i], conv_b2[i], batch3,
                               W_d1, b_d1, wo_p, bo_p)
    return out[:, :OUT]


# speedup vs baseline: 1.0491x; 1.0491x over previous
import jax
import jax.numpy as jnp
from jax import lax
from jax.experimental import pallas as pl
from jax.experimental.pallas import tpu as pltpu
from jax.experimental.pallas import tpu_sc as plsc

N = 10000
E = 320000
D = 128
G = 64
H1 = 256
OUT = 10
L = 3

NC = 2
NS = 16
NW = NC * NS
EPW = E // NW
C = 40
NCH = EPW // C
N_PAD = 10240
ROWS_PER_TILE = N_PAD // NS
NSTAGE = 5
NJ = NCH // NSTAGE

EPS = 1e-7


def _sc_body(h_hbm, e_hbm, src_hbm, dst_hbm, out_hbm,
             src_v, dst_v, hb0, hb1, hb2, hb3, eb0, eb1, acc,
             g0, g1, g2, g3, es0, es1, ss0, ss1, ss2, ss3):
    c = lax.axis_index("c")
    s = lax.axis_index("s")
    wid = c * NS + s
    hbufs = (hb0, hb1, hb2, hb3)
    ebufs = (eb0, eb1)
    gsems = (g0, g1, g2, g3)
    esems = (es0, es1)
    ssems = (ss0, ss1, ss2, ss3)

    def zrow(r, _):
        zero = jnp.zeros((16,), jnp.float32)
        for cc in range(8):
            hb0[r, pl.ds(cc * 16, 16)] = zero
        return 0
    lax.fori_loop(0, C, zrow, 0)
    for k in range(ROWS_PER_TILE // C):
        pltpu.sync_copy(hb0, acc.at[pl.ds(s * ROWS_PER_TILE + k * C, C)])
    plsc.subcore_barrier()

    ebase = wid * EPW

    def compute(hbuf, ebuf):
        def row(r, _):
            for cc in range(8):
                sl = pl.ds(cc * 16, 16)
                hbuf[r, sl] = jnp.maximum(hbuf[r, sl] + ebuf[r, sl], EPS)
            return 0
        lax.fori_loop(0, C, row, 0)

    def stage(st, _):
        pltpu.sync_copy(src_hbm.at[wid, st], src_v)
        pltpu.sync_copy(dst_hbm.at[wid, st], dst_v)
        gd = [None] * NJ
        ed = [None] * NJ
        sd = [None] * NJ
        for q in range(3):
            gd[q] = pltpu.async_copy(h_hbm.at[src_v.at[q]],
                                     hbufs[q], gsems[q])
        ed[0] = pltpu.async_copy(e_hbm.at[pl.ds((ebase + st * NJ * C), C)],
                                 eb0, es0)
        for j in range(NJ):
            b = j % 4
            if j + 3 < NJ:
                if j >= 1:
                    sd[j - 1].wait()
                gd[j + 3] = pltpu.async_copy(h_hbm.at[src_v.at[j + 3]],
                                             hbufs[(j + 3) % 4],
                                             gsems[(j + 3) % 4])
            if j + 1 < NJ:
                ed[j + 1] = pltpu.async_copy(
                    e_hbm.at[pl.ds(ebase + (st * NJ + j + 1) * C, C)],
                    ebufs[(j + 1) & 1], esems[(j + 1) & 1])
            gd[j].wait()
            ed[j].wait()
            compute(hbufs[b], ebufs[j & 1])
            sd[j] = pltpu.async_copy(hbufs[b], acc.at[dst_v.at[j]],
                                     ssems[b], add=True)
        for q in range(4):
            sd[NJ - 1 - q].wait()
        return 0
    lax.fori_loop(0, NSTAGE, stage, 0)

    plsc.subcore_barrier()
    pltpu.sync_copy(acc.at[pl.ds(s * ROWS_PER_TILE, ROWS_PER_TILE)],
                    out_hbm.at[pl.ds(c * N_PAD + s * ROWS_PER_TILE, ROWS_PER_TILE)])


@jax.jit
def _sc_msg_agg(h, e_eps, src_r, dst_r):
    mesh = plsc.VectorSubcoreMesh(core_axis_name="c", subcore_axis_name="s",
                                  num_cores=NC, num_subcores=NS)
    return pl.kernel(
        _sc_body,
        out_type=jax.ShapeDtypeStruct((NC * N_PAD, D), jnp.float32),
        mesh=mesh,
        scratch_types=[
            pltpu.VMEM((NJ, C), jnp.int32),
            pltpu.VMEM((NJ, C), jnp.int32),
            pltpu.VMEM((C, D), jnp.float32),
            pltpu.VMEM((C, D), jnp.float32),
            pltpu.VMEM((C, D), jnp.float32),
            pltpu.VMEM((C, D), jnp.float32),
            pltpu.VMEM((C, D), jnp.float32),
            pltpu.VMEM((C, D), jnp.float32),
            pltpu.VMEM_SHARED((N_PAD, D), jnp.float32),
            pltpu.SemaphoreType.DMA,
            pltpu.SemaphoreType.DMA,
            pltpu.SemaphoreType.DMA,
            pltpu.SemaphoreType.DMA,
            pltpu.SemaphoreType.DMA,
            pltpu.SemaphoreType.DMA,
            pltpu.SemaphoreType.DMA,
            pltpu.SemaphoreType.DMA,
            pltpu.SemaphoreType.DMA,
            pltpu.SemaphoreType.DMA,
        ],
    )(h, e_eps, src_r, dst_r)


_EBLK = 2000
_HBLK = 1024


def _enc_both_body(ea_ref, we_ref, be_ref, x_ref, wn_ref, bn_ref,
                   e_ref, h_ref):
    i = pl.program_id(0)
    e_ref[...] = jnp.dot(ea_ref[...], we_ref[...],
                         preferred_element_type=jnp.float32) + be_ref[...]

    @pl.when(i < N_PAD // _HBLK)
    def _node():
        h_ref[...] = jnp.dot(x_ref[...], wn_ref[...],
                             preferred_element_type=jnp.float32) + bn_ref[...]


def _tc_encode_both(edge_attr, we, be, x_pad, wn, bn):
    k_e = edge_attr.shape[1]
    nh = N_PAD // _HBLK
    return pl.pallas_call(
        _enc_both_body,
        grid=(E // _EBLK,),
        in_specs=[
            pl.BlockSpec((_EBLK, k_e), lambda i: (i, 0)),
            pl.BlockSpec((k_e, D), lambda i: (0, 0)),
            pl.BlockSpec((1, D), lambda i: (0, 0)),
            pl.BlockSpec((_HBLK, D), lambda i: (jnp.minimum(i, nh - 1), 0)),
            pl.BlockSpec((D, D), lambda i: (0, 0)),
            pl.BlockSpec((1, D), lambda i: (0, 0)),
        ],
        out_specs=[
            pl.BlockSpec((_EBLK, D), lambda i: (i, 0)),
            pl.BlockSpec((_HBLK, D), lambda i: (jnp.minimum(i, nh - 1), 0)),
        ],
        out_shape=[jax.ShapeDtypeStruct((E, D), jnp.float32),
                   jax.ShapeDtypeStruct((N_PAD, D), jnp.float32)],
    )(edge_attr, we, (be + EPS).reshape(1, D), x_pad, wn, bn.reshape(1, D))


def _mlp_body(p_ref, h_ref, w1_ref, b1_ref, w2_ref, b2_ref, o_ref):
    a = p_ref[0] + p_ref[1] + h_ref[...]
    mid = jnp.maximum(jnp.dot(a, w1_ref[...],
                              preferred_element_type=jnp.float32) + b1_ref[...], 0.0)
    o = jnp.dot(mid, w2_ref[...], preferred_element_type=jnp.float32) + b2_ref[...]
    o_ref[...] = jnp.maximum(o, 0.0)


def _tc_mlp(parts, h, w1, b1, w2, b2):
    blk = 1024
    return pl.pallas_call(
        _mlp_body,
        grid=(N_PAD // blk,),
        in_specs=[
            pl.BlockSpec((NC, blk, D), lambda i: (0, i, 0)),
            pl.BlockSpec((blk, D), lambda i: (i, 0)),
            pl.BlockSpec((D, H1), lambda i: (0, 0)),
            pl.BlockSpec((1, H1), lambda i: (0, 0)),
            pl.BlockSpec((H1, D), lambda i: (0, 0)),
            pl.BlockSpec((1, D), lambda i: (0, 0)),
        ],
        out_specs=pl.BlockSpec((blk, D), lambda i: (i, 0)),
        out_shape=jax.ShapeDtypeStruct((N_PAD, D), jnp.float32),
    )(parts, h, w1, b1.reshape(1, H1), w2, b2.reshape(1, D))


def _mlp_head_body(p_ref, h_ref, w1_ref, b1_ref, w2_ref, b2_ref, b3_ref,
                   wd1_ref, bd1_ref, wo_ref, bo_ref, o_ref, sums, counts):
    i = pl.program_id(0)
    nb = pl.num_programs(0)

    @pl.when(i == 0)
    def _init():
        sums[...] = jnp.zeros_like(sums)
        counts[...] = jnp.zeros_like(counts)

    a = p_ref[0] + p_ref[1] + h_ref[...]
    mid = jnp.maximum(jnp.dot(a, w1_ref[...],
                              preferred_element_type=jnp.float32) + b1_ref[...], 0.0)
    o = jnp.dot(mid, w2_ref[...], preferred_element_type=jnp.float32) + b2_ref[...]
    o = jnp.maximum(o, 0.0)

    bblk = b3_ref[0]
    gids = lax.broadcasted_iota(jnp.int32, (G, bblk.shape[1]), 0)
    oh = (gids == bblk).astype(jnp.float32)
    sums[...] += jnp.dot(oh, o, preferred_element_type=jnp.float32)
    counts[...] += jnp.dot(oh, jnp.ones_like(o),
                           preferred_element_type=jnp.float32)

    @pl.when(i == nb - 1)
    def _fin():
        pooled = sums[...] / jnp.maximum(counts[...], 1.0)
        z = jnp.maximum(jnp.dot(pooled, wd1_ref[...],
                                preferred_element_type=jnp.float32) + bd1_ref[...], 0.0)
        o_ref[...] = jnp.dot(z, wo_ref[...],
                             preferred_element_type=jnp.float32) + bo_ref[...]


def _tc_mlp_head(parts, h, w1, b1, w2, b2, batch3, wd1, bd1, wo_p, bo_p):
    blk = 1024
    return pl.pallas_call(
        _mlp_head_body,
        grid=(N_PAD // blk,),
        in_specs=[
            pl.BlockSpec((NC, blk, D), lambda i: (0, i, 0)),
            pl.BlockSpec((blk, D), lambda i: (i, 0)),
            pl.BlockSpec((D, H1), lambda i: (0, 0)),
            pl.BlockSpec((1, H1), lambda i: (0, 0)),
            pl.BlockSpec((H1, D), lambda i: (0, 0)),
            pl.BlockSpec((1, D), lambda i: (0, 0)),
            pl.BlockSpec((1, 1, blk), lambda i: (i, 0, 0)),
            pl.BlockSpec((D, H1), lambda i: (0, 0)),
            pl.BlockSpec((1, H1), lambda i: (0, 0)),
            pl.BlockSpec((H1, D), lambda i: (0, 0)),
            pl.BlockSpec((1, D), lambda i: (0, 0)),
        ],
        out_specs=pl.BlockSpec((G, D), lambda i: (0, 0)),
        out_shape=jax.ShapeDtypeStruct((G, D), jnp.float32),
        scratch_shapes=[
            pltpu.VMEM((G, D), jnp.float32),
            pltpu.VMEM((G, D), jnp.float32),
        ],
    )(parts, h, w1, b1.reshape(1, H1), w2, b2.reshape(1, D), batch3,
      wd1, bd1.reshape(1, H1), wo_p, bo_p)


def kernel(x, edge_index, edge_attr, batch, W_node, b_node, W_edge, b_edge,
           conv_W1, conv_b1, conv_W2, conv_b2, W_d1, b_d1, W_out, b_out):
    src_r = edge_index[0].reshape(NW, NSTAGE, NJ, C)
    dst_r = edge_index[1].reshape(NW, NSTAGE, NJ, C)
    x_pad = jnp.zeros((N_PAD, D), jnp.float32).at[:N].set(x)
    batch_pad = jnp.concatenate(
        [batch, jnp.full((N_PAD - N,), G, jnp.int32)])
    batch3 = batch_pad.reshape(N_PAD // 1024, 1, 1024)
    wo_p = jnp.zeros((H1, D), jnp.float32).at[:, :OUT].set(W_out)
    bo_p = jnp.zeros((1, D), jnp.float32).at[:, :OUT].set(b_out)

    e, h = _tc_encode_both(edge_attr, W_edge, b_edge, x_pad, W_node, b_node)
    for i in range(L):
        parts = _sc_msg_agg(h, e, src_r, dst_r).reshape(NC, N_PAD, D)
        if i < L - 1:
            h = _tc_mlp(parts, h, conv_W1[i], conv_b1[i],
                        conv_W2[i], conv_b2[i])
        else:
            out = _tc_mlp_head(parts, h, conv_W1[i], conv_b1[i],
                               conv_W2[i], conv_b2[i], batch3,
                               W_d1, b_d1, wo_p, bo_p)
    return out[:, :OUT]
